# baseline (speedup 1.0000x reference)
import functools

import jax
import jax.numpy as jnp
from jax import lax
from jax.experimental import pallas as pl
from jax.experimental.pallas import tpu as pltpu
from jax.experimental.pallas import tpu_sc as plsc

N = 10000
D = 128
E = 320000
NC = 2
NS = 16
NW = NC * NS
B = 128
NBATCH = 80
EPAD = NW * NBATCH * B
NROWS = EPAD // B
NACC = 10112
ZROWS = NACC // NS
ZCH = (128, 128, 128, 128, 120)
DEGW = 16
SB = 3
DB = 4

_mesh = plsc.VectorSubcoreMesh(
    core_axis_name="c", subcore_axis_name="s", num_cores=NC, num_subcores=NS
)


@functools.partial(
    pl.kernel,
    out_type=jax.ShapeDtypeStruct((NC, NACC, DEGW), jnp.float32),
    mesh=_mesh,
    scratch_types=[
        pltpu.VMEM((NBATCH, B), jnp.int32),
        pltpu.VMEM((B,), jnp.int32),
        pltpu.VMEM((B,), jnp.int32),
        pltpu.VMEM((B, DEGW), jnp.float32),
        pltpu.VMEM_SHARED((NACC, DEGW), jnp.float32),
        pltpu.SemaphoreType.DMA,
        pltpu.SemaphoreType.DMA,
    ],
)
def _deg_kernel(dst_hbm, out_hbm, didx, didxb0, didxb1, ones, acc, sem0, sem1):
    c = lax.axis_index("c")
    s = lax.axis_index("s")
    w = s * NC + c

    def fill(val):
        def body(j, _):
            ones[j, :] = jnp.full((DEGW,), val, jnp.float32)
            return 0
        lax.fori_loop(0, B, body, 0)

    fill(0.0)
    off = 0
    for cs in ZCH:
        base = pl.multiple_of(s * ZROWS + off, 8)
        pltpu.sync_copy(ones.at[pl.ds(0, cs)], acc.at[pl.ds(base, cs)])
        off += cs
    fill(1.0)
    pltpu.sync_copy(dst_hbm.at[pl.ds(pl.multiple_of(w * NBATCH, 8), NBATCH)], didx)
    plsc.subcore_barrier()

    dbufs = (didxb0, didxb1)
    ssems = (sem0, sem1)
    cps = [None] * NBATCH
    for i in range(NBATCH):
        p = i % 2
        for j in range(B // 16):
            dbufs[p][pl.ds(j * 16, 16)] = didx[i, pl.ds(j * 16, 16)]
        cps[i] = pltpu.async_copy(ones, acc.at[dbufs[p]], ssems[p], add=True)
        if i - 1 >= 0:
            cps[i - 1].wait()
    cps[NBATCH - 1].wait()
    plsc.subcore_barrier()
    off = 0
    for cs in ZCH:
        base = pl.multiple_of(s * ZROWS + off, 8)
        pltpu.sync_copy(acc.at[pl.ds(base, cs)], ones.at[pl.ds(0, cs)])
        pltpu.sync_copy(ones.at[pl.ds(0, cs)], out_hbm.at[c, pl.ds(base, cs)])
        off += cs


@functools.partial(
    pl.kernel,
    out_type=jax.ShapeDtypeStruct((NC, NACC, D), jnp.float32),
    mesh=_mesh,
    scratch_types=[
        pltpu.VMEM((SB, B, D), jnp.float32),
        pltpu.VMEM((B,), jnp.int32),
        pltpu.VMEM((B,), jnp.int32),
        pltpu.VMEM((B,), jnp.int32),
        pltpu.VMEM((B,), jnp.int32),
        pltpu.VMEM((B,), jnp.int32),
        pltpu.VMEM((B,), jnp.int32),
        pltpu.VMEM((B,), jnp.int32),
        pltpu.VMEM_SHARED((NACC, D), jnp.float32),
    ]
    + [pltpu.SemaphoreType.DMA] * (3 * SB + DB),
)
def _agg_kernel(
    hp_hbm, src_hbm, dst_hbm, out_hbm,
    bufs, si0, si1, si2, di0, di1, di2, di3, acc, *sems
):
    gsems = sems[:SB]
    ssems = sems[SB:2 * SB]
    isems = sems[2 * SB:3 * SB]
    dsems = sems[3 * SB:]
    sbufs = (si0, si1, si2)
    dbufs = (di0, di1, di2, di3)
    c = lax.axis_index("c")
    s = lax.axis_index("s")
    w = s * NC + c

    def zero(i, _):
        for j in range(D // 16):
            bufs[0, i, pl.ds(j * 16, 16)] = jnp.zeros((16,), jnp.float32)
        return 0

    lax.fori_loop(0, B, zero, 0)
    off = 0
    for cs in ZCH:
        base = pl.multiple_of(s * ZROWS + off, 8)
        pltpu.sync_copy(bufs.at[0, pl.ds(0, cs)], acc.at[pl.ds(base, cs)])
        off += cs
    plsc.subcore_barrier()

    def start_i(i):
        boff = pl.multiple_of((w * NBATCH + i) * B, B)
        return (
            pltpu.async_copy(src_hbm.at[pl.ds(boff, B)], sbufs[i % SB], isems[i % SB]),
            pltpu.async_copy(dst_hbm.at[pl.ds(boff, B)], dbufs[i % DB], dsems[i % DB]),
        )

    def start_g(i):
        p = i % SB
        return pltpu.async_copy(hp_hbm.at[sbufs[p]], bufs.at[p], gsems[p])

    def start_s(i):
        return pltpu.async_copy(
            bufs.at[i % SB], acc.at[dbufs[i % DB]], ssems[i % SB], add=True
        )

    NB = NBATCH
    cps_i = [None] * NB
    cps_g = [None] * NB
    cps_s = [None] * NB
    for i in range(min(3, NB)):
        cps_i[i] = start_i(i)
    for i in range(min(2, NB)):
        cps_i[i][0].wait()
        cps_i[i][1].wait()
        cps_g[i] = start_g(i)
    for i in range(NB):
        cps_g[i].wait()
        cps_s[i] = start_s(i)
        if i + 2 < NB:
            if i - 1 >= 0:
                cps_s[i - 1].wait()
            cps_i[i + 2][0].wait()
            cps_i[i + 2][1].wait()
            cps_g[i + 2] = start_g(i + 2)
        if i + 3 < NB:
            cps_i[i + 3] = start_i(i + 3)
    for i in range(NB - 3, NB):
        cps_s[i].wait()
    plsc.subcore_barrier()
    off = 0
    for cs in ZCH:
        base = pl.multiple_of(s * ZROWS + off, 8)
        pltpu.sync_copy(acc.at[pl.ds(base, cs)], bufs.at[0, pl.ds(0, cs)])
        pltpu.sync_copy(bufs.at[0, pl.ds(0, cs)], out_hbm.at[c, pl.ds(base, cs)])
        off += cs


_RB = 1000


def _tc1_body(x_ref, w_ref, deg_ref, hp_ref, dinv_ref):
    deg = deg_ref[0] + deg_ref[1] + 1.0
    dinv = lax.rsqrt(deg)
    dinv_ref[...] = dinv
    scale = dinv[:, 0:1]
    hp_ref[...] = (
        jnp.dot(x_ref[...], w_ref[...], preferred_element_type=jnp.float32)
        * scale
    )


def _tc2_body(agg_ref, hp_ref, dinv_ref, b_ref, w_ref, out_ref):
    ssum = agg_ref[0] + agg_ref[1] + hp_ref[...]
    scale = dinv_ref[...][:, 0:1]
    h1 = jnp.maximum(ssum * scale + b_ref[...], 0.0)
    out_ref[...] = (
        jnp.dot(h1, w_ref[...], preferred_element_type=jnp.float32) * scale
    )


def _tc3_body(agg_ref, hp_ref, dinv_ref, b_ref, out_ref):
    ssum = agg_ref[0] + agg_ref[1] + hp_ref[...]
    scale = dinv_ref[...][:, 0:1]
    out_ref[...] = jnp.maximum(ssum * scale + b_ref[...], 0.0)


_tc1 = pl.pallas_call(
    _tc1_body,
    grid=(N // _RB,),
    in_specs=[
        pl.BlockSpec((_RB, D), lambda i: (i, 0)),
        pl.BlockSpec((D, D), lambda i: (0, 0)),
        pl.BlockSpec((NC, _RB, DEGW), lambda i: (0, i, 0)),
    ],
    out_specs=[
        pl.BlockSpec((_RB, D), lambda i: (i, 0)),
        pl.BlockSpec((_RB, DEGW), lambda i: (i, 0)),
    ],
    out_shape=[
        jax.ShapeDtypeStruct((N, D), jnp.float32),
        jax.ShapeDtypeStruct((N, DEGW), jnp.float32),
    ],
)

_tc2 = pl.pallas_call(
    _tc2_body,
    grid=(N // _RB,),
    in_specs=[
        pl.BlockSpec((NC, _RB, D), lambda i: (0, i, 0)),
        pl.BlockSpec((_RB, D), lambda i: (i, 0)),
        pl.BlockSpec((_RB, DEGW), lambda i: (i, 0)),
        pl.BlockSpec((1, D), lambda i: (0, 0)),
        pl.BlockSpec((D, D), lambda i: (0, 0)),
    ],
    out_specs=pl.BlockSpec((_RB, D), lambda i: (i, 0)),
    out_shape=jax.ShapeDtypeStruct((N, D), jnp.float32),
)

_tc3 = pl.pallas_call(
    _tc3_body,
    grid=(N // _RB,),
    in_specs=[
        pl.BlockSpec((NC, _RB, D), lambda i: (0, i, 0)),
        pl.BlockSpec((_RB, D), lambda i: (i, 0)),
        pl.BlockSpec((_RB, DEGW), lambda i: (i, 0)),
        pl.BlockSpec((1, D), lambda i: (0, 0)),
    ],
    out_specs=pl.BlockSpec((_RB, D), lambda i: (i, 0)),
    out_shape=jax.ShapeDtypeStruct((N, D), jnp.float32),
)


@jax.jit
def kernel(x, edge_index, W1, b1, W2, b2):
    src = edge_index[0].astype(jnp.int32)
    dst = edge_index[1].astype(jnp.int32)
    pad = EPAD - E
    src_p = jnp.concatenate([src, jnp.zeros((pad,), jnp.int32)])
    dst_p = jnp.concatenate([dst, jnp.full((pad,), N, jnp.int32)])

    degp = _deg_kernel(dst_p.reshape(NROWS, B))
    hp1, dinv16 = _tc1(x, W1, degp)
    agg1 = _agg_kernel(hp1, src_p, dst_p)
    h2p = _tc2(agg1, hp1, dinv16, b1.reshape(1, D), W2)
    agg2 = _agg_kernel(h2p, src_p, dst_p)
    return _tc3(agg2, h2p, dinv16, b2.reshape(1, D))


# speedup vs baseline: 1.0264x; 1.0264x over previous
import functools

import jax
import jax.numpy as jnp
from jax import lax
from jax.experimental import pallas as pl
from jax.experimental.pallas import tpu as pltpu
from jax.experimental.pallas import tpu_sc as plsc

N = 10000
D = 128
E = 320000
NC = 2
NS = 16
NW = NC * NS
B = 128
NBATCH = 80
EPAD = NW * NBATCH * B
NROWS = EPAD // B
NACC = 10112
ZROWS = NACC // NS
ZCH = (128, 128, 128, 128, 120)
DEGW = 16
SB = 2

_mesh = plsc.VectorSubcoreMesh(
    core_axis_name="c", subcore_axis_name="s", num_cores=NC, num_subcores=NS
)


@functools.partial(
    pl.kernel,
    out_type=jax.ShapeDtypeStruct((NC, NACC, DEGW), jnp.float32),
    mesh=_mesh,
    scratch_types=[
        pltpu.VMEM((B,), jnp.int32),
        pltpu.VMEM((B,), jnp.int32),
        pltpu.VMEM((B, DEGW), jnp.float32),
        pltpu.VMEM_SHARED((NACC, DEGW), jnp.float32),
        pltpu.SemaphoreType.DMA,
        pltpu.SemaphoreType.DMA,
        pltpu.SemaphoreType.DMA,
        pltpu.SemaphoreType.DMA,
    ],
)
def _deg_kernel(dst_hbm, out_hbm, didxb0, didxb1, ones, acc, is0, is1, ss0, ss1):
    c = lax.axis_index("c")
    s = lax.axis_index("s")
    w = s * NC + c

    def fill(val):
        def body(j, _):
            ones[j, :] = jnp.full((DEGW,), val, jnp.float32)
            return 0
        lax.fori_loop(0, B, body, 0)

    fill(0.0)
    off = 0
    for cs in ZCH:
        base = pl.multiple_of(s * ZROWS + off, 8)
        pltpu.sync_copy(ones.at[pl.ds(0, cs)], acc.at[pl.ds(base, cs)])
        off += cs
    fill(1.0)
    plsc.subcore_barrier()

    dbufs = (didxb0, didxb1)
    isems = (is0, is1)
    ssems = (ss0, ss1)
    KB = 2
    NIT = NBATCH // KB
    base_off = w * NBATCH * B

    for p in range(KB):
        pltpu.async_copy(
            dst_hbm.at[pl.ds(pl.multiple_of(base_off + p * B, B), B)],
            dbufs[p], isems[p],
        )

    def body(j, _):
        for p in range(KB):
            pltpu.make_async_copy(dst_hbm.at[pl.ds(0, B)], dbufs[p], isems[p]).wait()
            pltpu.async_copy(ones, acc.at[dbufs[p]], ssems[p], add=True)
        for p in range(KB):
            pltpu.make_async_copy(ones, acc.at[dbufs[p]], ssems[p]).wait()

        @pl.when(j < NIT - 1)
        def _():
            for p in range(KB):
                boff = pl.multiple_of(base_off + ((j + 1) * KB + p) * B, B)
                pltpu.async_copy(dst_hbm.at[pl.ds(boff, B)], dbufs[p], isems[p])

        return 0

    lax.fori_loop(0, NIT, body, 0)
    plsc.subcore_barrier()
    off = 0
    for cs in ZCH:
        base = pl.multiple_of(s * ZROWS + off, 8)
        pltpu.sync_copy(acc.at[pl.ds(base, cs)], ones.at[pl.ds(0, cs)])
        pltpu.sync_copy(ones.at[pl.ds(0, cs)], out_hbm.at[c, pl.ds(base, cs)])
        off += cs


@functools.partial(
    pl.kernel,
    out_type=jax.ShapeDtypeStruct((NC, NACC, D), jnp.float32),
    mesh=_mesh,
    scratch_types=[
        pltpu.VMEM((SB, B, D), jnp.float32),
        pltpu.VMEM((B,), jnp.int32),
        pltpu.VMEM((B,), jnp.int32),
        pltpu.VMEM((B,), jnp.int32),
        pltpu.VMEM((B,), jnp.int32),
        pltpu.VMEM_SHARED((NACC, D), jnp.float32),
    ]
    + [pltpu.SemaphoreType.DMA] * (4 * SB),
)
def _agg_kernel(
    hp_hbm, src_hbm, dst_hbm, out_hbm,
    bufs, si0, si1, di0, di1, acc, *sems
):
    gsems = sems[:SB]
    ssems = sems[SB:2 * SB]
    isems = sems[2 * SB:3 * SB]
    dsems = sems[3 * SB:]
    sbufs = (si0, si1)
    dbufs = (di0, di1)
    c = lax.axis_index("c")
    s = lax.axis_index("s")
    w = s * NC + c

    def zero(i, _):
        for j in range(D // 16):
            bufs[0, i, pl.ds(j * 16, 16)] = jnp.zeros((16,), jnp.float32)
        return 0

    lax.fori_loop(0, B, zero, 0)
    off = 0
    for cs in ZCH:
        base = pl.multiple_of(s * ZROWS + off, 8)
        pltpu.sync_copy(bufs.at[0, pl.ds(0, cs)], acc.at[pl.ds(base, cs)])
        off += cs
    plsc.subcore_barrier()

    KB = 2
    NIT = NBATCH // KB
    base_off = w * NBATCH * B

    def fire_idx(boff, p):
        pltpu.async_copy(src_hbm.at[pl.ds(boff, B)], sbufs[p], isems[p])
        pltpu.async_copy(dst_hbm.at[pl.ds(boff, B)], dbufs[p], dsems[p])

    def wait_idx(p):
        pltpu.make_async_copy(src_hbm.at[pl.ds(0, B)], sbufs[p], isems[p]).wait()
        pltpu.make_async_copy(dst_hbm.at[pl.ds(0, B)], dbufs[p], dsems[p]).wait()

    def fire_g(p):
        pltpu.async_copy(hp_hbm.at[sbufs[p]], bufs.at[p], gsems[p])

    def wait_g(p):
        pltpu.make_async_copy(hp_hbm.at[sbufs[p]], bufs.at[p], gsems[p]).wait()

    def fire_s(p):
        pltpu.async_copy(bufs.at[p], acc.at[dbufs[p]], ssems[p], add=True)

    def wait_s(p):
        pltpu.make_async_copy(bufs.at[p], acc.at[dbufs[p]], ssems[p]).wait()

    for p in range(KB):
        fire_idx(pl.multiple_of(base_off + p * B, B), p)
    for p in range(KB):
        wait_idx(p)
        fire_g(p)

    def body(j, _):
        for p in range(KB):
            wait_g(p)
            fire_s(p)
        for p in range(KB):
            wait_s(p)

        @pl.when(j < NIT - 1)
        def _():
            for p in range(KB):
                boff = pl.multiple_of(base_off + ((j + 1) * KB + p) * B, B)
                fire_idx(boff, p)
            for p in range(KB):
                wait_idx(p)
                fire_g(p)

        return 0

    lax.fori_loop(0, NIT, body, 0)
    plsc.subcore_barrier()
    off = 0
    for cs in ZCH:
        base = pl.multiple_of(s * ZROWS + off, 8)
        pltpu.sync_copy(acc.at[pl.ds(base, cs)], bufs.at[0, pl.ds(0, cs)])
        pltpu.sync_copy(bufs.at[0, pl.ds(0, cs)], out_hbm.at[c, pl.ds(base, cs)])
        off += cs


_RB = 1000


def _tc1_body(x_ref, w_ref, deg_ref, hp_ref, dinv_ref):
    deg = deg_ref[0] + deg_ref[1] + 1.0
    dinv = lax.rsqrt(deg)
    dinv_ref[...] = dinv
    scale = dinv[:, 0:1]
    hp_ref[...] = (
        jnp.dot(x_ref[...], w_ref[...], preferred_element_type=jnp.float32)
        * scale
    )


def _tc2_body(agg_ref, hp_ref, dinv_ref, b_ref, w_ref, out_ref):
    ssum = agg_ref[0] + agg_ref[1] + hp_ref[...]
    scale = dinv_ref[...][:, 0:1]
    h1 = jnp.maximum(ssum * scale + b_ref[...], 0.0)
    out_ref[...] = (
        jnp.dot(h1, w_ref[...], preferred_element_type=jnp.float32) * scale
    )


def _tc3_body(agg_ref, hp_ref, dinv_ref, b_ref, out_ref):
    ssum = agg_ref[0] + agg_ref[1] + hp_ref[...]
    scale = dinv_ref[...][:, 0:1]
    out_ref[...] = jnp.maximum(ssum * scale + b_ref[...], 0.0)


_tc1 = pl.pallas_call(
    _tc1_body,
    grid=(N // _RB,),
    in_specs=[
        pl.BlockSpec((_RB, D), lambda i: (i, 0)),
        pl.BlockSpec((D, D), lambda i: (0, 0)),
        pl.BlockSpec((NC, _RB, DEGW), lambda i: (0, i, 0)),
    ],
    out_specs=[
        pl.BlockSpec((_RB, D), lambda i: (i, 0)),
        pl.BlockSpec((_RB, DEGW), lambda i: (i, 0)),
    ],
    out_shape=[
        jax.ShapeDtypeStruct((N, D), jnp.float32),
        jax.ShapeDtypeStruct((N, DEGW), jnp.float32),
    ],
)

_tc2 = pl.pallas_call(
    _tc2_body,
    grid=(N // _RB,),
    in_specs=[
        pl.BlockSpec((NC, _RB, D), lambda i: (0, i, 0)),
        pl.BlockSpec((_RB, D), lambda i: (i, 0)),
        pl.BlockSpec((_RB, DEGW), lambda i: (i, 0)),
        pl.BlockSpec((1, D), lambda i: (0, 0)),
        pl.BlockSpec((D, D), lambda i: (0, 0)),
    ],
    out_specs=pl.BlockSpec((_RB, D), lambda i: (i, 0)),
    out_shape=jax.ShapeDtypeStruct((N, D), jnp.float32),
)

_tc3 = pl.pallas_call(
    _tc3_body,
    grid=(N // _RB,),
    in_specs=[
        pl.BlockSpec((NC, _RB, D), lambda i: (0, i, 0)),
        pl.BlockSpec((_RB, D), lambda i: (i, 0)),
        pl.BlockSpec((_RB, DEGW), lambda i: (i, 0)),
        pl.BlockSpec((1, D), lambda i: (0, 0)),
    ],
    out_specs=pl.BlockSpec((_RB, D), lambda i: (i, 0)),
    out_shape=jax.ShapeDtypeStruct((N, D), jnp.float32),
)


@jax.jit
def kernel(x, edge_index, W1, b1, W2, b2):
    src = edge_index[0].astype(jnp.int32)
    dst = edge_index[1].astype(jnp.int32)
    pad = EPAD - E
    src_p = jnp.concatenate([src, jnp.zeros((pad,), jnp.int32)])
    dst_p = jnp.concatenate([dst, jnp.full((pad,), N, jnp.int32)])

    degp = _deg_kernel(dst_p)
    hp1, dinv16 = _tc1(x, W1, degp)
    agg1 = _agg_kernel(hp1, src_p, dst_p)
    h2p = _tc2(agg1, hp1, dinv16, b1.reshape(1, D), W2)
    agg2 = _agg_kernel(h2p, src_p, dst_p)
    return _tc3(agg2, h2p, dinv16, b2.reshape(1, D))
